# initial kernel scaffold (unmeasured)
import jax
import jax.numpy as jnp
from jax import lax
from jax.experimental import pallas as pl
from jax.experimental.pallas import tpu as pltpu

N_DEV = 16


def kernel(x, w_mat):
    m_per, k = x.shape
    _, n_per = w_mat.shape

    def body(x_ref, w_ref, out_ref, buf, w_bf, amax_buf,
             send_sems, recv_sems, asend_sems, arecv_sems):
        my = lax.axis_index("i")
        left = (my - 1) % N_DEV
        right = (my + 1) % N_DEV

        barrier = pltpu.get_barrier_semaphore()
        pl.semaphore_signal(barrier, 1, device_id=(left,),
                            device_id_type=pl.DeviceIdType.MESH)
        pl.semaphore_signal(barrier, 1, device_id=(right,),
                            device_id_type=pl.DeviceIdType.MESH)
        pl.semaphore_wait(barrier, 2)

        w_bf[...] = w_ref[...].astype(jnp.bfloat16)
        buf[pl.ds(my * m_per, m_per), :] = x_ref[...].astype(jnp.bfloat16)

        def gemm(origin):
            a = buf[pl.ds(origin * m_per, m_per), :]
            out_ref[pl.ds(origin * m_per, m_per), :] = jnp.dot(
                a, w_bf[...], preferred_element_type=jnp.float32)

        for h in range(N_DEV - 1):
            orig_s = (my - h) % N_DEV
            orig_r = (my - h - 1) % N_DEV
            send = pltpu.make_async_remote_copy(
                src_ref=buf.at[pl.ds(orig_s * m_per, m_per)],
                dst_ref=buf.at[pl.ds(orig_s * m_per, m_per)],
                send_sem=send_sems.at[orig_s],
                recv_sem=recv_sems.at[orig_s],
                device_id=(right,),
                device_id_type=pl.DeviceIdType.MESH,
            )
            send.start()
            gemm(orig_s)
            recv = pltpu.make_async_remote_copy(
                src_ref=buf.at[pl.ds(orig_r * m_per, m_per)],
                dst_ref=buf.at[pl.ds(orig_r * m_per, m_per)],
                send_sem=send_sems.at[orig_r],
                recv_sem=recv_sems.at[orig_r],
                device_id=(left,),
                device_id_type=pl.DeviceIdType.MESH,
            )
            recv.wait_recv()
            send.wait_send()
        gemm((my + 1) % N_DEV)

        amax_local = jnp.max(jnp.abs(out_ref[...]))
        amax_buf[pl.ds(my * 8, 8), :] = jnp.full((8, 128), amax_local,
                                                 jnp.float32)
        for d in range(1, N_DEV):
            tgt = (my + d) % N_DEV
            pltpu.make_async_remote_copy(
                src_ref=amax_buf.at[pl.ds(my * 8, 8)],
                dst_ref=amax_buf.at[pl.ds(my * 8, 8)],
                send_sem=asend_sems.at[tgt],
                recv_sem=arecv_sems.at[my],
                device_id=(tgt,),
                device_id_type=pl.DeviceIdType.MESH,
            ).start()
        for d in range(1, N_DEV):
            src = (my + d) % N_DEV
            pltpu.make_async_remote_copy(
                src_ref=amax_buf.at[pl.ds(my * 8, 8)],
                dst_ref=amax_buf.at[pl.ds(src * 8, 8)],
                send_sem=asend_sems.at[src],
                recv_sem=arecv_sems.at[src],
                device_id=(src,),
                device_id_type=pl.DeviceIdType.MESH,
            ).wait_recv()
        for d in range(1, N_DEV):
            tgt = (my + d) % N_DEV
            pltpu.make_async_remote_copy(
                src_ref=amax_buf.at[pl.ds(my * 8, 8)],
                dst_ref=amax_buf.at[pl.ds(my * 8, 8)],
                send_sem=asend_sems.at[tgt],
                recv_sem=arecv_sems.at[my],
                device_id=(tgt,),
                device_id_type=pl.DeviceIdType.MESH,
            ).wait_send()

        g = jnp.max(amax_buf[...])
        scale = g / 127.0
        q = jnp.clip(jnp.rint(out_ref[...] / scale), -127.0, 127.0)
        out_ref[...] = q * scale

    return pl.pallas_call(
        body,
        out_shape=jax.ShapeDtypeStruct((N_DEV * m_per, n_per), jnp.float32),
        in_specs=[
            pl.BlockSpec(memory_space=pltpu.VMEM),
            pl.BlockSpec(memory_space=pltpu.VMEM),
        ],
        out_specs=pl.BlockSpec(memory_space=pltpu.VMEM),
        scratch_shapes=[
            pltpu.VMEM((N_DEV * m_per, k), jnp.bfloat16),
            pltpu.VMEM((k, n_per), jnp.bfloat16),
            pltpu.VMEM((N_DEV * 8, 128), jnp.float32),
            pltpu.SemaphoreType.DMA((N_DEV,)),
            pltpu.SemaphoreType.DMA((N_DEV,)),
            pltpu.SemaphoreType.DMA((N_DEV,)),
            pltpu.SemaphoreType.DMA((N_DEV,)),
        ],
        compiler_params=pltpu.CompilerParams(collective_id=0),
    )(x, w_mat)


# baseline (device time: 382856 ns/iter reference)
import jax
import jax.numpy as jnp
from jax import lax
from jax.experimental import pallas as pl
from jax.experimental.pallas import tpu as pltpu

N_DEV = 16


def kernel(x, w_mat):
    m_per, k = x.shape
    _, n_per = w_mat.shape

    def body(x_ref, w_ref, out_ref, buf, w_bf, amax_buf,
             send_sems, recv_sems, asend_sems, arecv_sems):
        my = lax.axis_index("i")
        left = (my - 1) % N_DEV
        right = (my + 1) % N_DEV

        barrier = pltpu.get_barrier_semaphore()
        pl.semaphore_signal(barrier, 1, device_id=(left,),
                            device_id_type=pl.DeviceIdType.MESH)
        pl.semaphore_signal(barrier, 1, device_id=(right,),
                            device_id_type=pl.DeviceIdType.MESH)
        pl.semaphore_wait(barrier, 2)

        w_bf[...] = w_ref[...].astype(jnp.bfloat16)
        buf[pl.ds(my * m_per, m_per), :] = x_ref[...].astype(jnp.bfloat16)

        def gemm(origin):
            a = buf[pl.ds(origin * m_per, m_per), :]
            out_ref[pl.ds(origin * m_per, m_per), :] = jnp.dot(
                a, w_bf[...], preferred_element_type=jnp.float32)

        for h in range(N_DEV - 1):
            orig_s = (my - h) % N_DEV
            orig_r = (my - h - 1) % N_DEV
            send = pltpu.make_async_remote_copy(
                src_ref=buf.at[pl.ds(orig_s * m_per, m_per)],
                dst_ref=buf.at[pl.ds(orig_s * m_per, m_per)],
                send_sem=send_sems.at[orig_s],
                recv_sem=recv_sems.at[orig_s],
                device_id=(right,),
                device_id_type=pl.DeviceIdType.MESH,
            )
            send.start()
            gemm(orig_s)
            recv = pltpu.make_async_remote_copy(
                src_ref=buf.at[pl.ds(orig_r * m_per, m_per)],
                dst_ref=buf.at[pl.ds(orig_r * m_per, m_per)],
                send_sem=send_sems.at[orig_r],
                recv_sem=recv_sems.at[orig_r],
                device_id=(left,),
                device_id_type=pl.DeviceIdType.MESH,
            )
            recv.wait_recv()
            send.wait_send()
        gemm((my + 1) % N_DEV)

        amax_local = jnp.max(jnp.abs(out_ref[...]))
        amax_buf[pl.ds(my * 8, 8), :] = jnp.full((8, 128), amax_local,
                                                 jnp.float32)
        for d in range(1, N_DEV):
            tgt = (my + d) % N_DEV
            pltpu.make_async_remote_copy(
                src_ref=amax_buf.at[pl.ds(my * 8, 8)],
                dst_ref=amax_buf.at[pl.ds(my * 8, 8)],
                send_sem=asend_sems.at[tgt],
                recv_sem=arecv_sems.at[my],
                device_id=(tgt,),
                device_id_type=pl.DeviceIdType.MESH,
            ).start()
        for d in range(1, N_DEV):
            src = (my + d) % N_DEV
            pltpu.make_async_remote_copy(
                src_ref=amax_buf.at[pl.ds(my * 8, 8)],
                dst_ref=amax_buf.at[pl.ds(src * 8, 8)],
                send_sem=asend_sems.at[src],
                recv_sem=arecv_sems.at[src],
                device_id=(src,),
                device_id_type=pl.DeviceIdType.MESH,
            ).wait_recv()
        for d in range(1, N_DEV):
            tgt = (my + d) % N_DEV
            pltpu.make_async_remote_copy(
                src_ref=amax_buf.at[pl.ds(my * 8, 8)],
                dst_ref=amax_buf.at[pl.ds(my * 8, 8)],
                send_sem=asend_sems.at[tgt],
                recv_sem=arecv_sems.at[my],
                device_id=(tgt,),
                device_id_type=pl.DeviceIdType.MESH,
            ).wait_send()

        g = jnp.max(amax_buf[...])
        scale = g / 127.0
        q = jnp.clip(jnp.rint(out_ref[...] / scale), -127.0, 127.0)
        out_ref[...] = q * scale

    return pl.pallas_call(
        body,
        out_shape=jax.ShapeDtypeStruct((N_DEV * m_per, n_per), jnp.float32),
        in_specs=[
            pl.BlockSpec(memory_space=pltpu.VMEM),
            pl.BlockSpec(memory_space=pltpu.VMEM),
        ],
        out_specs=pl.BlockSpec(memory_space=pltpu.VMEM),
        scratch_shapes=[
            pltpu.VMEM((N_DEV * m_per, k), jnp.bfloat16),
            pltpu.VMEM((k, n_per), jnp.bfloat16),
            pltpu.VMEM((N_DEV * 8, 128), jnp.float32),
            pltpu.SemaphoreType.DMA((N_DEV,)),
            pltpu.SemaphoreType.DMA((N_DEV,)),
            pltpu.SemaphoreType.DMA((N_DEV,)),
            pltpu.SemaphoreType.DMA((N_DEV,)),
        ],
        compiler_params=pltpu.CompilerParams(
            collective_id=0,
            vmem_limit_bytes=100 * 1024 * 1024,
        ),
    )(x, w_mat)


# device time: 217970 ns/iter; 1.7565x vs baseline; 1.7565x over previous
import jax
import jax.numpy as jnp
from jax import lax
from jax.experimental import pallas as pl
from jax.experimental.pallas import tpu as pltpu

N_DEV = 16


def kernel(x, w_mat):
    m_per, k = x.shape
    _, n_per = w_mat.shape
    half = m_per // 2

    def b_next(o):
        return jnp.where(o % 2 == 1, o - 1, (o + 3) % N_DEV)

    def b_prev(o):
        return jnp.where(o % 2 == 0, o + 1, (o + N_DEV - 3) % N_DEV)

    def body(x_ref, w_ref, out_ref, buf, w_bf, amax_buf,
             send_semsA, recv_semsA, send_semsB, recv_semsB,
             asend_sems, arecv_sems):
        my = lax.axis_index("i")
        a_left = (my - 1) % N_DEV
        a_right = (my + 1) % N_DEV
        b_left = b_prev(my)
        b_right = b_next(my)

        barrier = pltpu.get_barrier_semaphore()
        for nbr in (a_left, a_right, b_left, b_right):
            pl.semaphore_signal(barrier, 1, device_id=(nbr,),
                                device_id_type=pl.DeviceIdType.MESH)
        pl.semaphore_wait(barrier, 4)

        w_bf[...] = w_ref[...].astype(jnp.bfloat16)
        buf[pl.ds(my * m_per, m_per), :] = x_ref[...].astype(jnp.bfloat16)

        def gemm_rows(row0, nrows):
            a = buf[pl.ds(row0, nrows), :]
            out_ref[pl.ds(row0, nrows), :] = jnp.dot(
                a, w_bf[...], preferred_element_type=jnp.float32)

        gemm_rows(my * m_per, m_per)

        oa = my
        ob = my
        for h in range(N_DEV - 1):
            oa_r = (my - h - 1) % N_DEV
            ob_r = b_prev(ob)
            sendA = pltpu.make_async_remote_copy(
                src_ref=buf.at[pl.ds(oa * m_per, half)],
                dst_ref=buf.at[pl.ds(oa * m_per, half)],
                send_sem=send_semsA.at[oa],
                recv_sem=recv_semsA.at[oa],
                device_id=(a_right,),
                device_id_type=pl.DeviceIdType.MESH,
            )
            sendB = pltpu.make_async_remote_copy(
                src_ref=buf.at[pl.ds(ob * m_per + half, half)],
                dst_ref=buf.at[pl.ds(ob * m_per + half, half)],
                send_sem=send_semsB.at[ob],
                recv_sem=recv_semsB.at[ob],
                device_id=(b_right,),
                device_id_type=pl.DeviceIdType.MESH,
            )
            sendA.start()
            sendB.start()
            if h > 0:
                gemm_rows(oa * m_per, half)
                gemm_rows(ob * m_per + half, half)
            recvA = pltpu.make_async_remote_copy(
                src_ref=buf.at[pl.ds(oa_r * m_per, half)],
                dst_ref=buf.at[pl.ds(oa_r * m_per, half)],
                send_sem=send_semsA.at[oa_r],
                recv_sem=recv_semsA.at[oa_r],
                device_id=(a_left,),
                device_id_type=pl.DeviceIdType.MESH,
            )
            recvB = pltpu.make_async_remote_copy(
                src_ref=buf.at[pl.ds(ob_r * m_per + half, half)],
                dst_ref=buf.at[pl.ds(ob_r * m_per + half, half)],
                send_sem=send_semsB.at[ob_r],
                recv_sem=recv_semsB.at[ob_r],
                device_id=(b_left,),
                device_id_type=pl.DeviceIdType.MESH,
            )
            recvA.wait_recv()
            recvB.wait_recv()
            sendA.wait_send()
            sendB.wait_send()
            oa = oa_r
            ob = ob_r
        gemm_rows(oa * m_per, half)
        gemm_rows(ob * m_per + half, half)

        amax_local = jnp.max(jnp.abs(out_ref[...]))
        amax_buf[pl.ds(my * 8, 8), :] = jnp.full((8, 128), amax_local,
                                                 jnp.float32)
        for d in range(1, N_DEV):
            tgt = (my + d) % N_DEV
            pltpu.make_async_remote_copy(
                src_ref=amax_buf.at[pl.ds(my * 8, 8)],
                dst_ref=amax_buf.at[pl.ds(my * 8, 8)],
                send_sem=asend_sems.at[tgt],
                recv_sem=arecv_sems.at[my],
                device_id=(tgt,),
                device_id_type=pl.DeviceIdType.MESH,
            ).start()
        for d in range(1, N_DEV):
            src = (my + d) % N_DEV
            pltpu.make_async_remote_copy(
                src_ref=amax_buf.at[pl.ds(my * 8, 8)],
                dst_ref=amax_buf.at[pl.ds(src * 8, 8)],
                send_sem=asend_sems.at[src],
                recv_sem=arecv_sems.at[src],
                device_id=(src,),
                device_id_type=pl.DeviceIdType.MESH,
            ).wait_recv()
        for d in range(1, N_DEV):
            tgt = (my + d) % N_DEV
            pltpu.make_async_remote_copy(
                src_ref=amax_buf.at[pl.ds(my * 8, 8)],
                dst_ref=amax_buf.at[pl.ds(my * 8, 8)],
                send_sem=asend_sems.at[tgt],
                recv_sem=arecv_sems.at[my],
                device_id=(tgt,),
                device_id_type=pl.DeviceIdType.MESH,
            ).wait_send()

        g = jnp.max(amax_buf[...])
        scale = g / 127.0
        q = jnp.clip(jnp.rint(out_ref[...] / scale), -127.0, 127.0)
        out_ref[...] = q * scale

    return pl.pallas_call(
        body,
        out_shape=jax.ShapeDtypeStruct((N_DEV * m_per, n_per), jnp.float32),
        in_specs=[
            pl.BlockSpec(memory_space=pltpu.VMEM),
            pl.BlockSpec(memory_space=pltpu.VMEM),
        ],
        out_specs=pl.BlockSpec(memory_space=pltpu.VMEM),
        scratch_shapes=[
            pltpu.VMEM((N_DEV * m_per, k), jnp.bfloat16),
            pltpu.VMEM((k, n_per), jnp.bfloat16),
            pltpu.VMEM((N_DEV * 8, 128), jnp.float32),
            pltpu.SemaphoreType.DMA((N_DEV,)),
            pltpu.SemaphoreType.DMA((N_DEV,)),
            pltpu.SemaphoreType.DMA((N_DEV,)),
            pltpu.SemaphoreType.DMA((N_DEV,)),
            pltpu.SemaphoreType.DMA((N_DEV,)),
            pltpu.SemaphoreType.DMA((N_DEV,)),
        ],
        compiler_params=pltpu.CompilerParams(
            collective_id=0,
            vmem_limit_bytes=100 * 1024 * 1024,
        ),
    )(x, w_mat)


# device time: 189100 ns/iter; 2.0246x vs baseline; 1.1527x over previous
import jax
import jax.numpy as jnp
from jax import lax
from jax.experimental import pallas as pl
from jax.experimental.pallas import tpu as pltpu

N_DEV = 16
S = 2


def kernel(x, w_mat):
    m_per, k = x.shape
    _, n_per = w_mat.shape
    half = m_per // 2
    sub = half // S

    def b_next(o):
        return jnp.where(o % 2 == 1, o - 1, (o + 3) % N_DEV)

    def b_prev(o):
        return jnp.where(o % 2 == 0, o + 1, (o + N_DEV - 3) % N_DEV)

    def body(x_ref, w_ref, out_ref, buf, w_bf, amax_buf,
             send_semsA, recv_semsA, send_semsB, recv_semsB,
             asend_sems, arecv_sems):
        my = lax.axis_index("i")
        a_left = (my - 1) % N_DEV
        a_right = (my + 1) % N_DEV
        b_left = b_prev(my)
        b_right = b_next(my)

        def mkA(o, si, dev):
            row = o * m_per + si * sub
            return pltpu.make_async_remote_copy(
                src_ref=buf.at[pl.ds(row, sub)],
                dst_ref=buf.at[pl.ds(row, sub)],
                send_sem=send_semsA.at[o * S + si],
                recv_sem=recv_semsA.at[o * S + si],
                device_id=(dev,),
                device_id_type=pl.DeviceIdType.MESH,
            )

        def mkB(o, si, dev):
            row = o * m_per + half + si * sub
            return pltpu.make_async_remote_copy(
                src_ref=buf.at[pl.ds(row, sub)],
                dst_ref=buf.at[pl.ds(row, sub)],
                send_sem=send_semsB.at[o * S + si],
                recv_sem=recv_semsB.at[o * S + si],
                device_id=(dev,),
                device_id_type=pl.DeviceIdType.MESH,
            )

        buf[pl.ds(my * m_per, m_per), :] = x_ref[...].astype(jnp.bfloat16)

        barrier = pltpu.get_barrier_semaphore()
        for nbr in (a_left, a_right, b_left, b_right):
            pl.semaphore_signal(barrier, 1, device_id=(nbr,),
                                device_id_type=pl.DeviceIdType.MESH)
        pl.semaphore_wait(barrier, 4)

        for si in range(S):
            mkA(my, si, a_right).start()
            mkB(my, si, b_right).start()

        w_bf[...] = w_ref[...].astype(jnp.bfloat16)

        def gemm_rows(row0, nrows):
            a = buf[pl.ds(row0, nrows), :]
            out_ref[pl.ds(row0, nrows), :] = jnp.dot(
                a, w_bf[...], preferred_element_type=jnp.float32)

        gemm_rows(my * m_per, m_per)

        ob = my
        for h in range(N_DEV - 1):
            oa_r = (my - h - 1) % N_DEV
            ob_r = b_prev(ob)
            last = h == N_DEV - 2
            for si in range(S):
                mkA(oa_r, si, a_left).wait_recv()
                if not last:
                    mkA(oa_r, si, a_right).start()
                mkB(ob_r, si, b_left).wait_recv()
                if not last:
                    mkB(ob_r, si, b_right).start()
            gemm_rows(oa_r * m_per, half)
            gemm_rows(ob_r * m_per + half, half)
            ob = ob_r

        ob = my
        for h in range(N_DEV - 1):
            oa = (my - h) % N_DEV
            for si in range(S):
                mkA(oa, si, a_right).wait_send()
                mkB(ob, si, b_right).wait_send()
            ob = b_prev(ob)

        amax_local = jnp.max(jnp.abs(out_ref[...]))
        amax_buf[pl.ds(my * 8, 8), :] = jnp.full((8, 128), amax_local,
                                                 jnp.float32)
        for d in range(1, N_DEV):
            tgt = (my + d) % N_DEV
            pltpu.make_async_remote_copy(
                src_ref=amax_buf.at[pl.ds(my * 8, 8)],
                dst_ref=amax_buf.at[pl.ds(my * 8, 8)],
                send_sem=asend_sems.at[tgt],
                recv_sem=arecv_sems.at[my],
                device_id=(tgt,),
                device_id_type=pl.DeviceIdType.MESH,
            ).start()
        for d in range(1, N_DEV):
            src = (my + d) % N_DEV
            pltpu.make_async_remote_copy(
                src_ref=amax_buf.at[pl.ds(my * 8, 8)],
                dst_ref=amax_buf.at[pl.ds(src * 8, 8)],
                send_sem=asend_sems.at[src],
                recv_sem=arecv_sems.at[src],
                device_id=(src,),
                device_id_type=pl.DeviceIdType.MESH,
            ).wait_recv()
        for d in range(1, N_DEV):
            tgt = (my + d) % N_DEV
            pltpu.make_async_remote_copy(
                src_ref=amax_buf.at[pl.ds(my * 8, 8)],
                dst_ref=amax_buf.at[pl.ds(my * 8, 8)],
                send_sem=asend_sems.at[tgt],
                recv_sem=arecv_sems.at[my],
                device_id=(tgt,),
                device_id_type=pl.DeviceIdType.MESH,
            ).wait_send()

        g = jnp.max(amax_buf[...])
        scale = g / 127.0
        q = jnp.clip(jnp.rint(out_ref[...] / scale), -127.0, 127.0)
        out_ref[...] = q * scale

    return pl.pallas_call(
        body,
        out_shape=jax.ShapeDtypeStruct((N_DEV * m_per, n_per), jnp.float32),
        in_specs=[
            pl.BlockSpec(memory_space=pltpu.VMEM),
            pl.BlockSpec(memory_space=pltpu.VMEM),
        ],
        out_specs=pl.BlockSpec(memory_space=pltpu.VMEM),
        scratch_shapes=[
            pltpu.VMEM((N_DEV * m_per, k), jnp.bfloat16),
            pltpu.VMEM((k, n_per), jnp.bfloat16),
            pltpu.VMEM((N_DEV * 8, 128), jnp.float32),
            pltpu.SemaphoreType.DMA((N_DEV * S,)),
            pltpu.SemaphoreType.DMA((N_DEV * S,)),
            pltpu.SemaphoreType.DMA((N_DEV * S,)),
            pltpu.SemaphoreType.DMA((N_DEV * S,)),
            pltpu.SemaphoreType.DMA((N_DEV,)),
            pltpu.SemaphoreType.DMA((N_DEV,)),
        ],
        compiler_params=pltpu.CompilerParams(
            collective_id=0,
            vmem_limit_bytes=100 * 1024 * 1024,
        ),
    )(x, w_mat)
